# baseline (speedup 1.0000x reference)
import jax
import jax.numpy as jnp
from jax import lax
from jax.experimental import pallas as pl
from jax.experimental.pallas import tpu as pltpu

_DN_T = (((1,), (1,)), ((), ()))
_CHUNKS = 4


def _gcn_body(x_ref, w1_ref, b1_ref, w2_ref, b2_ref, w3_ref, b3_ref, o_ref, acc_ref):
    i = pl.program_id(0)
    part = jnp.sum(x_ref[...], axis=1)

    @pl.when(i == 0)
    def _():
        acc_ref[...] = part

    @pl.when(i > 0)
    def _():
        acc_ref[...] = acc_ref[...] + part

    @pl.when(i == _CHUNKS - 1)
    def _():
        n_total = x_ref.shape[1] * _CHUNKS
        xm = acc_ref[...] * (1.0 / n_total)
        y = lax.dot_general(xm, w1_ref[...], _DN_T, preferred_element_type=jnp.float32)
        y = jnp.maximum(y + b1_ref[...], 0.0)
        y = lax.dot_general(y, w2_ref[...], _DN_T, preferred_element_type=jnp.float32)
        y = jnp.maximum(y + b2_ref[...], 0.0)
        y = lax.dot_general(y, w3_ref[...], _DN_T, preferred_element_type=jnp.float32)
        y = jnp.maximum(y + b3_ref[...], 0.0)
        o_ref[...] = jnp.broadcast_to(y[:, None, :], o_ref.shape)


def kernel(x, W1, b1, W2, b2, W3, b3):
    B, N, C = x.shape
    hid = W1.shape[0]
    out_dim = W3.shape[0]
    out_rows = 128
    nc = N // _CHUNKS
    const2 = lambda i: (0, 0)
    return pl.pallas_call(
        _gcn_body,
        grid=(_CHUNKS,),
        in_specs=[
            pl.BlockSpec((B, nc, C), lambda i: (0, i, 0)),
            pl.BlockSpec((hid, C), const2),
            pl.BlockSpec((1, hid), const2),
            pl.BlockSpec((hid, hid), const2),
            pl.BlockSpec((1, hid), const2),
            pl.BlockSpec((out_dim, hid), const2),
            pl.BlockSpec((1, out_dim), const2),
        ],
        out_specs=pl.BlockSpec((B, out_rows, out_dim), lambda i: (0, 0, 0)),
        out_shape=jax.ShapeDtypeStruct((B, out_rows, out_dim), x.dtype),
        scratch_shapes=[pltpu.VMEM((B, C), jnp.float32)],
    )(
        x,
        W1, b1.reshape(1, -1),
        W2, b2.reshape(1, -1),
        W3, b3.reshape(1, -1),
    )


# speedup vs baseline: 1.1409x; 1.1409x over previous
import jax
import jax.numpy as jnp
from jax import lax
from jax.experimental import pallas as pl

_DN_T = (((1,), (1,)), ((), ()))


def _gcn_body(x_ref, w1_ref, b1_ref, w2_ref, b2_ref, w3_ref, b3_ref, o_ref):
    x = x_ref[...]
    n = x.shape[1]
    xm = jnp.sum(x, axis=1) * (1.0 / n)
    y = lax.dot_general(xm, w1_ref[...], _DN_T, preferred_element_type=jnp.float32)
    y = jnp.maximum(y + b1_ref[...], 0.0)
    y = lax.dot_general(y, w2_ref[...], _DN_T, preferred_element_type=jnp.float32)
    y = jnp.maximum(y + b2_ref[...], 0.0)
    y = lax.dot_general(y, w3_ref[...], _DN_T, preferred_element_type=jnp.float32)
    y = jnp.maximum(y + b3_ref[...], 0.0)
    o_ref[...] = jnp.broadcast_to(y[:, None, :], o_ref.shape)


def kernel(x, W1, b1, W2, b2, W3, b3):
    B, N, C = x.shape
    out_dim = W3.shape[0]
    out_rows = 128
    return pl.pallas_call(
        _gcn_body,
        out_shape=jax.ShapeDtypeStruct((B, out_rows, out_dim), x.dtype),
    )(
        x,
        W1, b1.reshape(1, -1),
        W2, b2.reshape(1, -1),
        W3, b3.reshape(1, -1),
    )
